# Initial kernel scaffold; baseline (speedup 1.0000x reference)
import jax
import jax.numpy as jnp
from jax.experimental import pallas as pl


def kernel(query, key, value):
    raise NotImplementedError("write your pallas kernel here")


# speedup vs baseline: 1.0746x; 1.0746x over previous
import jax
import jax.numpy as jnp
import numpy as np
from jax.experimental import pallas as pl

_BLOCK = 32
_NLOCAL = 2
_STRIDE = 8
_TQ = 256
_SUPER = _STRIDE * _BLOCK
_LOCW = _TQ + 2 * _BLOCK


def _attn_kernel(q_ref, k_ref, v_ref, o_ref):
    t = pl.program_id(1)
    q = q_ref[0]
    k = k_ref[0]
    v = v_ref[0]
    S, E = k.shape
    temp = 1.0 / float(np.sqrt(E))

    ks = k.reshape(-1, _SUPER, E)[:, :_BLOCK, :].reshape(-1, E)
    vs = v.reshape(-1, _SUPER, E)[:, :_BLOCK, :].reshape(-1, E)

    start = jnp.clip(t * _TQ - _BLOCK, 0, S - _LOCW)
    kl = k_ref[0, pl.ds(start, _LOCW), :]
    vl = v_ref[0, pl.ds(start, _LOCW), :]

    dn = (((1,), (1,)), ((), ()))
    ss = jax.lax.dot_general(q, ks, dn,
                             preferred_element_type=jnp.float32) * temp
    sl = jax.lax.dot_general(q, kl, dn,
                             preferred_element_type=jnp.float32) * temp

    ns = ss.shape[1]
    bi_s = (jax.lax.broadcasted_iota(jnp.int32, (_TQ, ns), 0) + t * _TQ) // _BLOCK
    js = (jax.lax.broadcasted_iota(jnp.int32, (_TQ, ns), 1) // _BLOCK) * _STRIDE
    ss = jnp.where(jnp.abs(bi_s - js) >= _NLOCAL, ss, -1e30)

    bi_l = (jax.lax.broadcasted_iota(jnp.int32, (_TQ, _LOCW), 0) + t * _TQ) // _BLOCK
    jl = start // _BLOCK + jax.lax.broadcasted_iota(jnp.int32, (_TQ, _LOCW), 1) // _BLOCK
    sl = jnp.where(jnp.abs(bi_l - jl) < _NLOCAL, sl, -1e30)

    m = jnp.maximum(jnp.max(ss, axis=1), jnp.max(sl, axis=1))
    ps = jnp.exp(ss - m[:, None])
    plc = jnp.exp(sl - m[:, None])
    denom = jnp.sum(ps, axis=1) + jnp.sum(plc, axis=1)

    dv = (((1,), (0,)), ((), ()))
    out = jax.lax.dot_general(ps, vs, dv, preferred_element_type=jnp.float32)
    out = out + jax.lax.dot_general(plc, vl, dv,
                                    preferred_element_type=jnp.float32)
    o_ref[0] = out / denom[:, None]


def kernel(query, key, value):
    B, T, H, E = query.shape
    S = key.shape[1]
    q = jnp.transpose(query[0], (1, 0, 2))
    k = jnp.transpose(key[0], (1, 0, 2))
    v = jnp.transpose(value[0], (1, 0, 2))

    grid = (H, T // _TQ)
    out = pl.pallas_call(
        _attn_kernel,
        grid=grid,
        in_specs=[
            pl.BlockSpec((1, _TQ, E), lambda h, t: (h, t, 0)),
            pl.BlockSpec((1, S, E), lambda h, t: (h, 0, 0)),
            pl.BlockSpec((1, S, E), lambda h, t: (h, 0, 0)),
        ],
        out_specs=pl.BlockSpec((1, _TQ, E), lambda h, t: (h, t, 0)),
        out_shape=jax.ShapeDtypeStruct((H, T, E), jnp.float32),
    )(q, k, v)
    return jnp.transpose(out, (1, 0, 2))[None]
